# Initial kernel scaffold; baseline (speedup 1.0000x reference)
import jax
import jax.numpy as jnp
from jax.experimental import pallas as pl


def kernel(x, edge_index, W1, attn_l1, attn_r1, b1, W2, attn_l2, attn_r2, b2):
    raise NotImplementedError("write your pallas kernel here")


# speedup vs baseline: 18.5245x; 18.5245x over previous
import functools

import jax
import jax.numpy as jnp
from jax import lax
from jax.experimental import pallas as pl
from jax.experimental.pallas import tpu as pltpu
from jax.experimental.pallas import tpu_sc as plsc

N = 10000
F = 128
NP = 10240
NEG = 0.2
NC, NS = 2, 16
NWORK = NC * NS
E_RAW = 320000
E_TOT = E_RAW + N
BATCH = 64
PER_TILE = 10368
E_PAD = PER_TILE * NWORK
NBATCH = PER_TILE // BATCH
ROWS_PT = NP // NS
BLK = 256


def _rowmask(i, x):
    rows = i * BLK + lax.broadcasted_iota(jnp.int32, x.shape, 0)
    return jnp.where(rows < N, x, 0.0)


def _project_body(i, h, w_ref, attn_ref, feat_ref, elr_ref):
    f = jnp.dot(h, w_ref[...], preferred_element_type=jnp.float32)
    f = _rowmask(i, f)
    feat_ref[...] = f
    elr_ref[...] = lax.dot_general(
        attn_ref[...], f, (((1,), (1,)), ((), ())),
        preferred_element_type=jnp.float32)


def _tc_a_body(x_ref, w_ref, attn_ref, feat_ref, elr_ref):
    i = pl.program_id(0)
    _project_body(i, x_ref[...], w_ref, attn_ref, feat_ref, elr_ref)


def _combine(a0_ref, a1_ref, es_ref, b_ref):
    a = a0_ref[...] + a1_ref[...]
    s = lax.dot_general(
        es_ref[...], jnp.ones((NWORK, 1), jnp.float32),
        (((0,), (0,)), ((), ())), preferred_element_type=jnp.float32)
    return a / s + b_ref[0:1, :]


def _tc_b_body(a0_ref, a1_ref, es_ref, b_ref, w_ref, attn_ref,
               feat_ref, elr_ref):
    i = pl.program_id(0)
    h = jnp.maximum(_combine(a0_ref, a1_ref, es_ref, b_ref), 0.0)
    h = _rowmask(i, h)
    _project_body(i, h, w_ref, attn_ref, feat_ref, elr_ref)


def _tc_c_body(a0_ref, a1_ref, es_ref, b_ref, o_ref):
    o_ref[...] = _combine(a0_ref, a1_ref, es_ref, b_ref)


def _tc_front(xp, wT, attn):
    return pl.pallas_call(
        _tc_a_body,
        grid=(NP // BLK,),
        in_specs=[
            pl.BlockSpec((BLK, F), lambda i: (i, 0)),
            pl.BlockSpec((F, F), lambda i: (0, 0)),
            pl.BlockSpec((8, F), lambda i: (0, 0)),
        ],
        out_specs=[
            pl.BlockSpec((BLK, F), lambda i: (i, 0)),
            pl.BlockSpec((8, BLK), lambda i: (0, i)),
        ],
        out_shape=[
            jax.ShapeDtypeStruct((NP, F), jnp.float32),
            jax.ShapeDtypeStruct((8, NP), jnp.float32),
        ],
    )(xp, wT, attn)


def _tc_mid(msg, esums, b2d, wT, attn):
    return pl.pallas_call(
        _tc_b_body,
        grid=(NP // BLK,),
        in_specs=[
            pl.BlockSpec((BLK, F), lambda i: (i, 0)),
            pl.BlockSpec((BLK, F), lambda i: (i, 0)),
            pl.BlockSpec((NWORK, BLK), lambda i: (0, i)),
            pl.BlockSpec((8, F), lambda i: (0, 0)),
            pl.BlockSpec((F, F), lambda i: (0, 0)),
            pl.BlockSpec((8, F), lambda i: (0, 0)),
        ],
        out_specs=[
            pl.BlockSpec((BLK, F), lambda i: (i, 0)),
            pl.BlockSpec((8, BLK), lambda i: (0, i)),
        ],
        out_shape=[
            jax.ShapeDtypeStruct((NP, F), jnp.float32),
            jax.ShapeDtypeStruct((8, NP), jnp.float32),
        ],
    )(msg[0], msg[1], esums, b2d, wT, attn)


def _tc_back(msg, esums, b2d):
    ngrid = (N + BLK - 1) // BLK
    return pl.pallas_call(
        _tc_c_body,
        grid=(ngrid,),
        in_specs=[
            pl.BlockSpec((BLK, F), lambda i: (i, 0)),
            pl.BlockSpec((BLK, F), lambda i: (i, 0)),
            pl.BlockSpec((NWORK, BLK), lambda i: (0, i)),
            pl.BlockSpec((8, F), lambda i: (0, 0)),
        ],
        out_specs=pl.BlockSpec((BLK, F), lambda i: (i, 0)),
        out_shape=jax.ShapeDtypeStruct((N, F), jnp.float32),
    )(msg[0], msg[1], esums, b2d)


def _sc_body(feat_hbm, elr_hbm, src_hbm, dst_hbm, msg_hbm, esums_hbm,
             acc, el_v, er_v, esum_v, src_v, dst_v, rows_v, gsem):
    c = lax.axis_index("c")
    s = lax.axis_index("s")
    wid = c * NS + s
    base_row = s * ROWS_PT

    def zrow(i, _):
        for cc in range(F // 16):
            rows_v[0, i, pl.ds(cc * 16, 16)] = jnp.zeros((16,), jnp.float32)
        return 0
    lax.fori_loop(0, BATCH, zrow, 0)
    for k in range(ROWS_PT // BATCH):
        pltpu.sync_copy(rows_v.at[0],
                        acc.at[pl.ds(base_row + k * BATCH, BATCH)])

    def zes(i, _):
        esum_v[pl.ds(i * 16, 16)] = jnp.zeros((16,), jnp.float32)
        return 0
    lax.fori_loop(0, NP // 16, zes, 0)

    pltpu.sync_copy(elr_hbm.at[0], el_v)
    pltpu.sync_copy(elr_hbm.at[1], er_v)

    def mx(i, carry):
        ml, mr = carry
        sl = pl.ds(i * 16, 16)
        return jnp.maximum(ml, el_v[sl]), jnp.maximum(mr, er_v[sl])
    ml, mr = lax.fori_loop(
        0, NP // 16, mx,
        (jnp.full((16,), -1e30, jnp.float32),
         jnp.full((16,), -1e30, jnp.float32)))

    dnums = lax.GatherDimensionNumbers(
        offset_dims=(), collapsed_slice_dims=(0,), start_index_map=(0,))

    def lane_allmax(v):
        for sh in (1, 2, 4, 8):
            idx = lax.iota(jnp.int32, 16) ^ sh
            perm = lax.gather(
                v, idx[:, None], dnums, (1,),
                mode=lax.GatherScatterMode.PROMISE_IN_BOUNDS)
            v = jnp.maximum(v, perm)
        return v
    mm = lane_allmax(ml) + lane_allmax(mr)
    big_m = jnp.where(mm >= 0, mm, NEG * mm)

    plsc.subcore_barrier()

    ebase = wid * PER_TILE

    def batch_body(j, _):
        b0 = ebase + j * BATCH
        pltpu.sync_copy(src_hbm.at[pl.ds(b0, BATCH)], src_v.at[0])
        pltpu.sync_copy(dst_hbm.at[pl.ds(b0, BATCH)], dst_v.at[0])
        pltpu.async_copy(feat_hbm.at[src_v.at[0]], rows_v.at[0], gsem).wait()

        def grp(g, _):
            sl = pl.ds(g * 16, 16)
            s16 = src_v[0, sl]
            d16 = dst_v[0, sl]
            e = plsc.load_gather(el_v, [s16]) + plsc.load_gather(er_v, [d16])
            e = jnp.where(e >= 0, e, NEG * e)
            ee = jnp.exp(e - big_m)
            plsc.addupdate_scatter(esum_v, [d16], ee)
            for k in range(16):
                eei = ee[k]
                edge = g * 16 + k
                for cc in range(F // 16):
                    csl = pl.ds(cc * 16, 16)
                    rows_v[0, edge, csl] = rows_v[0, edge, csl] * eei
            return 0
        lax.fori_loop(0, BATCH // 16, grp, 0)
        pltpu.sync_copy(rows_v.at[0], acc.at[dst_v.at[0]], add=True)
        return 0

    lax.fori_loop(0, NBATCH, batch_body, 0)
    plsc.subcore_barrier()

    for k in range(ROWS_PT // BATCH):
        r0 = base_row + k * BATCH
        pltpu.sync_copy(acc.at[pl.ds(r0, BATCH)], rows_v.at[0])
        pltpu.sync_copy(rows_v.at[0], msg_hbm.at[c, pl.ds(r0, BATCH)])
    pltpu.sync_copy(esum_v, esums_hbm.at[wid])


def _sc_agg(featE, elr, srcs, dsts):
    mesh = plsc.VectorSubcoreMesh(core_axis_name="c", subcore_axis_name="s",
                                  num_cores=NC, num_subcores=NS)
    run = pl.kernel(
        _sc_body,
        out_type=[
            jax.ShapeDtypeStruct((NC, NP, F), jnp.float32),
            jax.ShapeDtypeStruct((NWORK, NP), jnp.float32),
        ],
        mesh=mesh,
        compiler_params=pltpu.CompilerParams(needs_layout_passes=False),
        scratch_types=[
            pltpu.VMEM_SHARED((NP, F), jnp.float32),
            pltpu.VMEM((NP,), jnp.float32),
            pltpu.VMEM((NP,), jnp.float32),
            pltpu.VMEM((NP,), jnp.float32),
            pltpu.VMEM((2, BATCH), jnp.int32),
            pltpu.VMEM((2, BATCH), jnp.int32),
            pltpu.VMEM((2, BATCH, F), jnp.float32),
            pltpu.SemaphoreType.DMA,
        ],
    )
    return run(featE, elr, srcs, dsts)


def kernel(x, edge_index, W1, attn_l1, attn_r1, b1, W2, attn_l2, attn_r2, b2):
    f32 = jnp.float32
    xp = jnp.pad(x.astype(f32), ((0, NP - N), (0, 0)))
    loop = jnp.arange(N, dtype=jnp.int32)
    padv = jnp.full((E_PAD - E_TOT,), N, jnp.int32)
    src = jnp.concatenate([edge_index[0], loop, padv])
    dst = jnp.concatenate([edge_index[1], loop, padv])

    def attn_pack(al, ar):
        a = jnp.zeros((8, F), f32)
        return a.at[0].set(al[0]).at[1].set(ar[0])

    attn1 = attn_pack(attn_l1, attn_r1)
    attn2 = attn_pack(attn_l2, attn_r2)
    b1d = jnp.broadcast_to(b1.reshape(1, F), (8, F))
    b2d = jnp.broadcast_to(b2.reshape(1, F), (8, F))

    featE1, elr1 = _tc_front(xp, W1.T, attn1)
    msg1, esum1 = _sc_agg(featE1, elr1, src, dst)
    featE2, elr2 = _tc_mid(msg1, esum1, b1d, W2.T, attn2)
    msg2, esum2 = _sc_agg(featE2, elr2, src, dst)
    return _tc_back(msg2, esum2, b2d)


# speedup vs baseline: 24.5789x; 1.3268x over previous
import functools

import jax
import jax.numpy as jnp
from jax import lax
from jax.experimental import pallas as pl
from jax.experimental.pallas import tpu as pltpu
from jax.experimental.pallas import tpu_sc as plsc

N = 10000
F = 128
NP = 10240
NEG = 0.2
NC, NS = 2, 16
NWORK = NC * NS
E_RAW = 320000
E_TOT = E_RAW + N
BATCH = 48
NBUF = 3
PER_TILE = 10368
E_PAD = PER_TILE * NWORK
NBATCH = PER_TILE // BATCH
NPA = 10112
ROWS_PT = NPA // NS
BLK = 256


def _rowmask(i, x):
    rows = i * BLK + lax.broadcasted_iota(jnp.int32, x.shape, 0)
    return jnp.where(rows < N, x, 0.0)


def _project_body(i, h, w_ref, attn_ref, feat_ref, elr_ref):
    f = jnp.dot(h, w_ref[...], preferred_element_type=jnp.float32)
    f = _rowmask(i, f)
    feat_ref[...] = f
    elr_ref[...] = lax.dot_general(
        attn_ref[...], f, (((1,), (1,)), ((), ())),
        preferred_element_type=jnp.float32)


def _tc_a_body(x_ref, w_ref, attn_ref, feat_ref, elr_ref):
    i = pl.program_id(0)
    _project_body(i, x_ref[...], w_ref, attn_ref, feat_ref, elr_ref)


def _combine(a0_ref, a1_ref, es_ref, b_ref):
    a = a0_ref[...] + a1_ref[...]
    s = lax.dot_general(
        es_ref[...], jnp.ones((NWORK, 1), jnp.float32),
        (((0,), (0,)), ((), ())), preferred_element_type=jnp.float32)
    return a / s + b_ref[0:1, :]


def _tc_b_body(a0_ref, a1_ref, es_ref, b_ref, w_ref, attn_ref,
               feat_ref, elr_ref):
    i = pl.program_id(0)
    h = jnp.maximum(_combine(a0_ref, a1_ref, es_ref, b_ref), 0.0)
    h = _rowmask(i, h)
    _project_body(i, h, w_ref, attn_ref, feat_ref, elr_ref)


def _tc_c_body(a0_ref, a1_ref, es_ref, b_ref, o_ref):
    o_ref[...] = _combine(a0_ref, a1_ref, es_ref, b_ref)


def _tc_front(xp, wT, attn):
    return pl.pallas_call(
        _tc_a_body,
        grid=(NP // BLK,),
        in_specs=[
            pl.BlockSpec((BLK, F), lambda i: (i, 0)),
            pl.BlockSpec((F, F), lambda i: (0, 0)),
            pl.BlockSpec((8, F), lambda i: (0, 0)),
        ],
        out_specs=[
            pl.BlockSpec((BLK, F), lambda i: (i, 0)),
            pl.BlockSpec((8, BLK), lambda i: (0, i)),
        ],
        out_shape=[
            jax.ShapeDtypeStruct((NP, F), jnp.float32),
            jax.ShapeDtypeStruct((8, NP), jnp.float32),
        ],
    )(xp, wT, attn)


def _tc_mid(msg, esums, b2d, wT, attn):
    return pl.pallas_call(
        _tc_b_body,
        grid=(NP // BLK,),
        in_specs=[
            pl.BlockSpec((BLK, F), lambda i: (i, 0)),
            pl.BlockSpec((BLK, F), lambda i: (i, 0)),
            pl.BlockSpec((NWORK, BLK), lambda i: (0, i)),
            pl.BlockSpec((8, F), lambda i: (0, 0)),
            pl.BlockSpec((F, F), lambda i: (0, 0)),
            pl.BlockSpec((8, F), lambda i: (0, 0)),
        ],
        out_specs=[
            pl.BlockSpec((BLK, F), lambda i: (i, 0)),
            pl.BlockSpec((8, BLK), lambda i: (0, i)),
        ],
        out_shape=[
            jax.ShapeDtypeStruct((NP, F), jnp.float32),
            jax.ShapeDtypeStruct((8, NP), jnp.float32),
        ],
    )(msg[0], msg[1], esums, b2d, wT, attn)


def _tc_back(msg, esums, b2d):
    ngrid = (N + BLK - 1) // BLK
    return pl.pallas_call(
        _tc_c_body,
        grid=(ngrid,),
        in_specs=[
            pl.BlockSpec((BLK, F), lambda i: (i, 0)),
            pl.BlockSpec((BLK, F), lambda i: (i, 0)),
            pl.BlockSpec((NWORK, BLK), lambda i: (0, i)),
            pl.BlockSpec((8, F), lambda i: (0, 0)),
        ],
        out_specs=pl.BlockSpec((BLK, F), lambda i: (i, 0)),
        out_shape=jax.ShapeDtypeStruct((N, F), jnp.float32),
    )(msg[0], msg[1], esums, b2d)


def _zero_chunks():
    out = [BATCH] * (ROWS_PT // BATCH)
    if ROWS_PT % BATCH:
        out.append(ROWS_PT % BATCH)
    return out


def _sc_body(feat_hbm, elr_hbm, src_hbm, dst_hbm, msg_hbm, esums_hbm,
             acc, el_v, er_v, esum_v, src_v, dst_v, rows_v,
             gsem0, gsem1, gsem2, ssem0, ssem1, ssem2):
    gsems = (gsem0, gsem1, gsem2)
    ssems = (ssem0, ssem1, ssem2)
    c = lax.axis_index("c")
    s = lax.axis_index("s")
    wid = c * NS + s
    base_row = s * ROWS_PT

    def zrow(i, _):
        for cc in range(F // 16):
            rows_v[0, i, pl.ds(cc * 16, 16)] = jnp.zeros((16,), jnp.float32)
        return 0
    lax.fori_loop(0, BATCH, zrow, 0)
    off = 0
    for nch in _zero_chunks():
        pltpu.sync_copy(rows_v.at[0, pl.ds(0, nch)],
                        acc.at[pl.ds(base_row + off, nch)])
        off += nch

    def zes(i, _):
        esum_v[pl.ds(i * 16, 16)] = jnp.zeros((16,), jnp.float32)
        return 0
    lax.fori_loop(0, NP // 16, zes, 0)

    pltpu.sync_copy(elr_hbm.at[0], el_v)
    pltpu.sync_copy(elr_hbm.at[1], er_v)

    def mx(i, carry):
        ml, mr = carry
        sl = pl.ds(i * 16, 16)
        return jnp.maximum(ml, el_v[sl]), jnp.maximum(mr, er_v[sl])
    ml, mr = lax.fori_loop(
        0, NP // 16, mx,
        (jnp.full((16,), -1e30, jnp.float32),
         jnp.full((16,), -1e30, jnp.float32)))

    dnums = lax.GatherDimensionNumbers(
        offset_dims=(), collapsed_slice_dims=(0,), start_index_map=(0,))

    def lane_allmax(v):
        for sh in (1, 2, 4, 8):
            idx = lax.iota(jnp.int32, 16) ^ sh
            perm = lax.gather(
                v, idx[:, None], dnums, (1,),
                mode=lax.GatherScatterMode.PROMISE_IN_BOUNDS)
            v = jnp.maximum(v, perm)
        return v
    mm = lane_allmax(ml) + lane_allmax(mr)
    big_m = jnp.where(mm >= 0, mm, NEG * mm)

    plsc.subcore_barrier()

    ebase = wid * PER_TILE

    def fetch(slot, j):
        b0 = ebase + j * BATCH
        pltpu.sync_copy(src_hbm.at[pl.ds(b0, BATCH)], src_v.at[slot])
        pltpu.sync_copy(dst_hbm.at[pl.ds(b0, BATCH)], dst_v.at[slot])
        pltpu.async_copy(feat_hbm.at[src_v.at[slot]], rows_v.at[slot],
                         gsems[slot])

    def wait_gather(slot):
        pltpu.make_async_copy(feat_hbm.at[src_v.at[slot]], rows_v.at[slot],
                              gsems[slot]).wait()

    def wait_scatter(slot):
        pltpu.make_async_copy(rows_v.at[slot], acc.at[dst_v.at[slot]],
                              ssems[slot]).wait()

    def compute(slot):
        for g in range(BATCH // 16):
            sl = pl.ds(g * 16, 16)
            s16 = src_v[slot, sl]
            d16 = dst_v[slot, sl]
            e = plsc.load_gather(el_v, [s16]) + plsc.load_gather(er_v, [d16])
            e = jnp.where(e >= 0, e, NEG * e)
            ee = jnp.exp(e - big_m)
            plsc.addupdate_scatter(esum_v, [d16], ee)
            for k in range(16):
                eei = ee[k]
                edge = g * 16 + k
                for cc in range(F // 16):
                    csl = pl.ds(cc * 16, 16)
                    rows_v[slot, edge, csl] = rows_v[slot, edge, csl] * eei

    def scatter(slot):
        pltpu.async_copy(rows_v.at[slot], acc.at[dst_v.at[slot]],
                         ssems[slot], add=True)

    def visit(v, b, first):
        p = v + (NBUF - 1)
        bp = (b + NBUF - 1) % NBUF

        def prefetch():
            if not first:
                wait_scatter(bp)
            fetch(bp, p)
        if isinstance(v, int):
            prefetch()
        else:
            pl.when(p < NBATCH)(prefetch)
        wait_gather(b)
        compute(b)
        scatter(b)

    for j in range(NBUF - 1):
        fetch(j, j)
    for j in range(NBUF):
        visit(j, j, j == 0)

    def round_body(r, _):
        v = r * NBUF
        for i in range(NBUF):
            visit(v + i, i, False)
        return 0
    lax.fori_loop(1, NBATCH // NBUF, round_body, 0)
    for j in range(NBUF):
        wait_scatter(j)
    plsc.subcore_barrier()

    off = 0
    for nch in _zero_chunks():
        r0 = base_row + off
        pltpu.sync_copy(acc.at[pl.ds(r0, nch)], rows_v.at[0, pl.ds(0, nch)])
        pltpu.sync_copy(rows_v.at[0, pl.ds(0, nch)], msg_hbm.at[c, pl.ds(r0, nch)])
        off += nch
    pltpu.sync_copy(esum_v, esums_hbm.at[wid])


def _sc_agg(featE, elr, srcs, dsts):
    mesh = plsc.VectorSubcoreMesh(core_axis_name="c", subcore_axis_name="s",
                                  num_cores=NC, num_subcores=NS)
    run = pl.kernel(
        _sc_body,
        out_type=[
            jax.ShapeDtypeStruct((NC, NPA, F), jnp.float32),
            jax.ShapeDtypeStruct((NWORK, NP), jnp.float32),
        ],
        mesh=mesh,
        compiler_params=pltpu.CompilerParams(needs_layout_passes=False),
        scratch_types=[
            pltpu.VMEM_SHARED((NPA, F), jnp.float32),
            pltpu.VMEM((NP,), jnp.float32),
            pltpu.VMEM((NP,), jnp.float32),
            pltpu.VMEM((NP,), jnp.float32),
            pltpu.VMEM((NBUF, BATCH), jnp.int32),
            pltpu.VMEM((NBUF, BATCH), jnp.int32),
            pltpu.VMEM((NBUF, BATCH, F), jnp.float32),
            pltpu.SemaphoreType.DMA,
            pltpu.SemaphoreType.DMA,
            pltpu.SemaphoreType.DMA,
            pltpu.SemaphoreType.DMA,
            pltpu.SemaphoreType.DMA,
            pltpu.SemaphoreType.DMA,
        ],
    )
    return run(featE, elr, srcs, dsts)


def kernel(x, edge_index, W1, attn_l1, attn_r1, b1, W2, attn_l2, attn_r2, b2):
    f32 = jnp.float32
    xp = jnp.pad(x.astype(f32), ((0, NP - N), (0, 0)))
    loop = jnp.arange(N, dtype=jnp.int32)
    padv = jnp.full((E_PAD - E_TOT,), N, jnp.int32)
    src = jnp.concatenate([edge_index[0], loop, padv])
    dst = jnp.concatenate([edge_index[1], loop, padv])

    def attn_pack(al, ar):
        a = jnp.zeros((8, F), f32)
        return a.at[0].set(al[0]).at[1].set(ar[0])

    attn1 = attn_pack(attn_l1, attn_r1)
    attn2 = attn_pack(attn_l2, attn_r2)
    b1d = jnp.broadcast_to(b1.reshape(1, F), (8, F))
    b2d = jnp.broadcast_to(b2.reshape(1, F), (8, F))

    featE1, elr1 = _tc_front(xp, W1.T, attn1)
    msg1, esum1 = _sc_agg(featE1, elr1, src, dst)
    featE2, elr2 = _tc_mid(msg1, esum1, b1d, W2.T, attn2)
    msg2, esum2 = _sc_agg(featE2, elr2, src, dst)
    return _tc_back(msg2, esum2, b2d)


# speedup vs baseline: 30.1337x; 1.2260x over previous
import jax
import jax.numpy as jnp
from jax import lax
from jax.experimental import pallas as pl
from jax.experimental.pallas import tpu as pltpu
from jax.experimental.pallas import tpu_sc as plsc

N = 10000
F = 128
NP = 10112
NEG = 0.2
NC, NS = 2, 16
NWORK = NC * NS
E_RAW = 320000
E_TOT = E_RAW + N
BATCH = 48
NBUF = 3
KWIN = 6
PER_TILE = 10368
E_PAD = PER_TILE * NWORK
NBATCH = PER_TILE // BATCH
NACC = 10016
ROWS_PT = 632
BLK = 128


def _rowmask(i, x):
    rows = i * BLK + lax.broadcasted_iota(jnp.int32, x.shape, 0)
    return jnp.where(rows < N, x, 0.0)


def _project_body(i, h, w_ref, attn_ref, feat_ref, elr_ref, mstat_ref):
    f = jnp.dot(h, w_ref[...], preferred_element_type=jnp.float32)
    f = _rowmask(i, f)
    feat_ref[...] = f
    elr = lax.dot_general(
        attn_ref[...], f, (((1,), (1,)), ((), ())),
        preferred_element_type=jnp.float32)
    elr_ref[...] = elr
    bmax = jnp.max(elr, axis=1, keepdims=True)

    @pl.when(i == 0)
    def _():
        mstat_ref[...] = jnp.full((8, F), -1e30, jnp.float32)
    mstat_ref[...] = jnp.maximum(mstat_ref[...],
                                 jnp.broadcast_to(bmax, (8, F)))


def _tc_a_body(x_ref, w_ref, attn_ref, feat_ref, elr_ref, mstat_ref):
    i = pl.program_id(0)
    _project_body(i, x_ref[...], w_ref, attn_ref, feat_ref, elr_ref,
                  mstat_ref)


def _combine(a0_ref, a1_ref, es_ref, b_ref):
    a = a0_ref[...] + a1_ref[...]
    s = lax.dot_general(
        es_ref[...], jnp.ones((NWORK, 1), jnp.float32),
        (((0,), (0,)), ((), ())), preferred_element_type=jnp.float32)
    return a / s + b_ref[0:1, :]


def _tc_b_body(a0_ref, a1_ref, es_ref, b_ref, w_ref, attn_ref,
               feat_ref, elr_ref, mstat_ref):
    i = pl.program_id(0)
    h = jnp.maximum(_combine(a0_ref, a1_ref, es_ref, b_ref), 0.0)
    h = _rowmask(i, h)
    _project_body(i, h, w_ref, attn_ref, feat_ref, elr_ref, mstat_ref)


def _tc_c_body(a0_ref, a1_ref, es_ref, b_ref, o_ref):
    o_ref[...] = _combine(a0_ref, a1_ref, es_ref, b_ref)


_PROJ_OUT_SPECS = [
    pl.BlockSpec((BLK, F), lambda i: (i, 0)),
    pl.BlockSpec((8, BLK), lambda i: (0, i)),
    pl.BlockSpec((8, F), lambda i: (0, 0)),
]
_PROJ_OUT_SHAPE = [
    jax.ShapeDtypeStruct((NP, F), jnp.float32),
    jax.ShapeDtypeStruct((8, NP), jnp.float32),
    jax.ShapeDtypeStruct((8, F), jnp.float32),
]


def _tc_front(xp, wT, attn):
    return pl.pallas_call(
        _tc_a_body,
        grid=(NP // BLK,),
        in_specs=[
            pl.BlockSpec((BLK, F), lambda i: (i, 0)),
            pl.BlockSpec((F, F), lambda i: (0, 0)),
            pl.BlockSpec((8, F), lambda i: (0, 0)),
        ],
        out_specs=_PROJ_OUT_SPECS,
        out_shape=_PROJ_OUT_SHAPE,
    )(xp, wT, attn)


def _tc_mid(msg, esums, b2d, wT, attn):
    return pl.pallas_call(
        _tc_b_body,
        grid=(NP // BLK,),
        in_specs=[
            pl.BlockSpec((BLK, F), lambda i: (i, 0)),
            pl.BlockSpec((BLK, F), lambda i: (i, 0)),
            pl.BlockSpec((NWORK, BLK), lambda i: (0, i)),
            pl.BlockSpec((8, F), lambda i: (0, 0)),
            pl.BlockSpec((F, F), lambda i: (0, 0)),
            pl.BlockSpec((8, F), lambda i: (0, 0)),
        ],
        out_specs=_PROJ_OUT_SPECS,
        out_shape=_PROJ_OUT_SHAPE,
    )(msg[0], msg[1], esums, b2d, wT, attn)


def _tc_back(msg, esums, b2d):
    ngrid = (N + BLK - 1) // BLK
    return pl.pallas_call(
        _tc_c_body,
        grid=(ngrid,),
        in_specs=[
            pl.BlockSpec((BLK, F), lambda i: (i, 0)),
            pl.BlockSpec((BLK, F), lambda i: (i, 0)),
            pl.BlockSpec((NWORK, BLK), lambda i: (0, i)),
            pl.BlockSpec((8, F), lambda i: (0, 0)),
        ],
        out_specs=pl.BlockSpec((BLK, F), lambda i: (i, 0)),
        out_shape=jax.ShapeDtypeStruct((N, F), jnp.float32),
    )(msg[0], msg[1], esums, b2d)


def _sc_body(feat_hbm, elr_hbm, mstat_hbm, srcw_hbm, dstw_hbm,
             msg_hbm, esums_hbm,
             acc, el_v, er_v, esum_v, srcw_v, dstw_v, dstb_v, rows_v, m_v,
             gsem0, gsem1, gsem2, ssem0, ssem1, ssem2):
    gsems = (gsem0, gsem1, gsem2)
    ssems = (ssem0, ssem1, ssem2)
    c = lax.axis_index("c")
    s = lax.axis_index("s")
    wid = c * NS + s
    base_row = s * ROWS_PT
    n48 = jnp.where(s < NS - 1, ROWS_PT // BATCH, (NACC - 15 * ROWS_PT - 8)
                    // BATCH)

    def zrow(i, _):
        for cc in range(F // 16):
            rows_v[0, i, pl.ds(cc * 16, 16)] = jnp.zeros((16,), jnp.float32)
        return 0
    lax.fori_loop(0, BATCH, zrow, 0)

    def zacc(i, _):
        pltpu.sync_copy(rows_v.at[0],
                        acc.at[pl.ds(base_row + i * BATCH, BATCH)])
        return 0
    lax.fori_loop(0, n48, zacc, 0)
    pltpu.sync_copy(rows_v.at[0, pl.ds(0, 8)],
                    acc.at[pl.ds(base_row + n48 * BATCH, 8)])

    def zes(i, _):
        esum_v[pl.ds(i * 16, 16)] = jnp.zeros((16,), jnp.float32)
        return 0
    lax.fori_loop(0, NP // 16, zes, 0)

    pltpu.sync_copy(elr_hbm.at[0], el_v)
    pltpu.sync_copy(elr_hbm.at[1], er_v)
    pltpu.sync_copy(mstat_hbm.at[0], m_v.at[pl.ds(0, F)])
    pltpu.sync_copy(mstat_hbm.at[1], m_v.at[pl.ds(F, F)])
    mm = m_v[pl.ds(0, 16)] + m_v[pl.ds(F, 16)]
    big_m = jnp.where(mm >= 0, mm, NEG * mm)

    plsc.subcore_barrier()

    WW = KWIN * BATCH
    ebase = wid * PER_TILE

    def fetch_window(wi):
        o = lax.rem(wi, 2) * WW
        g0 = ebase + wi * WW
        pltpu.sync_copy(srcw_hbm.at[pl.ds(g0, WW)], srcw_v.at[pl.ds(o, WW)])
        pltpu.sync_copy(dstw_hbm.at[pl.ds(g0, WW)], dstw_v.at[pl.ds(o, WW)])

    def woff(j):
        return (lax.rem(lax.div(j, KWIN), 2) * KWIN + lax.rem(j, KWIN)) * BATCH

    def start_gather(slot, j):
        pltpu.async_copy(feat_hbm.at[srcw_v.at[pl.ds(woff(j), BATCH)]],
                         rows_v.at[slot], gsems[slot])

    def wait_gather(slot):
        pltpu.make_async_copy(feat_hbm.at[srcw_v.at[pl.ds(0, BATCH)]],
                              rows_v.at[slot], gsems[slot]).wait()

    def wait_scatter(slot):
        pltpu.make_async_copy(rows_v.at[slot], acc.at[dstb_v.at[slot]],
                              ssems[slot]).wait()

    def compute(slot, j):
        o = woff(j)
        for g in range(BATCH // 16):
            sl = pl.ds(o + g * 16, 16)
            s16 = srcw_v[sl]
            d16 = dstw_v[sl]
            dstb_v[slot, pl.ds(g * 16, 16)] = d16
            e = plsc.load_gather(el_v, [s16]) + plsc.load_gather(er_v, [d16])
            e = jnp.where(e >= 0, e, NEG * e)
            ee = jnp.exp(e - big_m)
            plsc.addupdate_scatter(esum_v, [d16], ee)
            for k in range(16):
                eei = ee[k]
                edge = g * 16 + k
                for cc in range(F // 16):
                    csl = pl.ds(cc * 16, 16)
                    rows_v[slot, edge, csl] = rows_v[slot, edge, csl] * eei

    def scatter(slot, j):
        pltpu.async_copy(rows_v.at[slot], acc.at[dstb_v.at[slot]],
                         ssems[slot], add=True)

    def visit(v, b, first):
        p = v + (NBUF - 1)

        def prefetch():
            bp = (b + NBUF - 1) % NBUF
            pl.when(lax.rem(jnp.int32(p), KWIN) == 0)(
                lambda: fetch_window(lax.div(jnp.int32(p), KWIN)))
            if not first:
                wait_scatter(bp)
            start_gather(bp, p)
        if isinstance(v, int):
            prefetch()
        else:
            pl.when(p < NBATCH)(prefetch)
        wait_gather(b)
        compute(b, v)
        scatter(b, v)

    fetch_window(0)
    for j in range(NBUF - 1):
        start_gather(j, j)
    for j in range(NBUF):
        visit(j, j, j == 0)

    def round_body(r, _):
        v = r * NBUF
        for i in range(NBUF):
            visit(v + i, i, False)
        return 0
    lax.fori_loop(1, NBATCH // NBUF, round_body, 0)
    for j in range(NBUF):
        wait_scatter(j)
    plsc.subcore_barrier()

    def wb(i, _):
        r0 = base_row + i * BATCH
        pltpu.sync_copy(acc.at[pl.ds(r0, BATCH)], rows_v.at[0])
        pltpu.sync_copy(rows_v.at[0], msg_hbm.at[c, pl.ds(r0, BATCH)])
        return 0
    lax.fori_loop(0, n48, wb, 0)
    rt = base_row + n48 * BATCH
    pltpu.sync_copy(acc.at[pl.ds(rt, 8)], rows_v.at[0, pl.ds(0, 8)])
    pltpu.sync_copy(rows_v.at[0, pl.ds(0, 8)], msg_hbm.at[c, pl.ds(rt, 8)])
    pltpu.sync_copy(esum_v, esums_hbm.at[wid])


def _sc_agg(featE, elr, mstat, srcw, dstw):
    mesh = plsc.VectorSubcoreMesh(core_axis_name="c", subcore_axis_name="s",
                                  num_cores=NC, num_subcores=NS)
    run = pl.kernel(
        _sc_body,
        out_type=[
            jax.ShapeDtypeStruct((NC, NACC, F), jnp.float32),
            jax.ShapeDtypeStruct((NWORK, NP), jnp.float32),
        ],
        mesh=mesh,
        compiler_params=pltpu.CompilerParams(needs_layout_passes=False),
        scratch_types=[
            pltpu.VMEM_SHARED((NACC, F), jnp.float32),
            pltpu.VMEM((NP,), jnp.float32),
            pltpu.VMEM((NP,), jnp.float32),
            pltpu.VMEM((NP,), jnp.float32),
            pltpu.VMEM((2 * KWIN * BATCH,), jnp.int32),
            pltpu.VMEM((2 * KWIN * BATCH,), jnp.int32),
            pltpu.VMEM((NBUF, BATCH), jnp.int32),
            pltpu.VMEM((NBUF, BATCH, F), jnp.float32),
            pltpu.VMEM((2 * F,), jnp.float32),
            pltpu.SemaphoreType.DMA,
            pltpu.SemaphoreType.DMA,
            pltpu.SemaphoreType.DMA,
            pltpu.SemaphoreType.DMA,
            pltpu.SemaphoreType.DMA,
            pltpu.SemaphoreType.DMA,
        ],
    )
    return run(featE, elr, mstat, srcw, dstw)


def kernel(x, edge_index, W1, attn_l1, attn_r1, b1, W2, attn_l2, attn_r2, b2):
    f32 = jnp.float32
    xp = jnp.pad(x.astype(f32), ((0, NP - N), (0, 0)))
    loop = jnp.arange(N, dtype=jnp.int32)
    padv = jnp.full((E_PAD - E_TOT,), N, jnp.int32)
    srcw = jnp.concatenate([edge_index[0], loop, padv])
    dstw = jnp.concatenate([edge_index[1], loop, padv])

    def attn_pack(al, ar):
        a = jnp.zeros((8, F), f32)
        return a.at[0].set(al[0]).at[1].set(ar[0])

    attn1 = attn_pack(attn_l1, attn_r1)
    attn2 = attn_pack(attn_l2, attn_r2)
    b1d = jnp.broadcast_to(b1.reshape(1, F), (8, F))
    b2d = jnp.broadcast_to(b2.reshape(1, F), (8, F))

    featE1, elr1, mstat1 = _tc_front(xp, W1.T, attn1)
    msg1, esum1 = _sc_agg(featE1, elr1, mstat1, srcw, dstw)
    featE2, elr2, mstat2 = _tc_mid(msg1, esum1, b1d, W2.T, attn2)
    msg2, esum2 = _sc_agg(featE2, elr2, mstat2, srcw, dstw)
    return _tc_back(msg2, esum2, b2d)


# speedup vs baseline: 32.1023x; 1.0653x over previous
import jax
import jax.numpy as jnp
from jax import lax
from jax.experimental import pallas as pl
from jax.experimental.pallas import tpu as pltpu
from jax.experimental.pallas import tpu_sc as plsc

N = 10000
F = 128
NP = 10112
NEG = 0.2
NC, NS = 2, 16
NWORK = NC * NS
E_RAW = 320000
E_TOT = E_RAW + N
BATCH = 48
NBUF = 3
KWIN = 6
PER_TILE = 10368
E_PAD = PER_TILE * NWORK
NBATCH = PER_TILE // BATCH
NACC = 10016
ROWS_PT = 632
BLK = 128


def _rowmask(i, x):
    rows = i * BLK + lax.broadcasted_iota(jnp.int32, x.shape, 0)
    return jnp.where(rows < N, x, 0.0)


def _project_body(i, h, w_ref, attn_ref, feat_ref, elr_ref, mstat_ref):
    f = jnp.dot(h, w_ref[...], preferred_element_type=jnp.float32)
    f = _rowmask(i, f)
    feat_ref[...] = f
    elr = lax.dot_general(
        attn_ref[...], f, (((1,), (1,)), ((), ())),
        preferred_element_type=jnp.float32)
    elr_ref[...] = elr
    bmax = jnp.max(elr, axis=1, keepdims=True)

    @pl.when(i == 0)
    def _():
        mstat_ref[...] = jnp.full((8, F), -1e30, jnp.float32)
    mstat_ref[...] = jnp.maximum(mstat_ref[...],
                                 jnp.broadcast_to(bmax, (8, F)))


def _tc_a_body(x_ref, w_ref, attn_ref, feat_ref, elr_ref, mstat_ref):
    i = pl.program_id(0)
    _project_body(i, x_ref[...], w_ref, attn_ref, feat_ref, elr_ref,
                  mstat_ref)


def _combine(a0_ref, a1_ref, es_ref, b_ref):
    a = a0_ref[...] + a1_ref[...]
    s = lax.dot_general(
        es_ref[...], jnp.ones((NWORK, 1), jnp.float32),
        (((0,), (0,)), ((), ())), preferred_element_type=jnp.float32)
    return a / s + b_ref[0:1, :]


def _tc_b_body(a0_ref, a1_ref, es_ref, b_ref, w_ref, attn_ref,
               feat_ref, elr_ref, mstat_ref):
    i = pl.program_id(0)
    h = jnp.maximum(_combine(a0_ref, a1_ref, es_ref, b_ref), 0.0)
    h = _rowmask(i, h)
    _project_body(i, h, w_ref, attn_ref, feat_ref, elr_ref, mstat_ref)


def _tc_c_body(a0_ref, a1_ref, es_ref, b_ref, o_ref):
    o_ref[...] = _combine(a0_ref, a1_ref, es_ref, b_ref)


_PROJ_OUT_SPECS = [
    pl.BlockSpec((BLK, F), lambda i: (i, 0)),
    pl.BlockSpec((8, BLK), lambda i: (0, i)),
    pl.BlockSpec((8, F), lambda i: (0, 0)),
]
_PROJ_OUT_SHAPE = [
    jax.ShapeDtypeStruct((NP, F), jnp.float32),
    jax.ShapeDtypeStruct((8, NP), jnp.float32),
    jax.ShapeDtypeStruct((8, F), jnp.float32),
]


def _tc_front(xp, wT, attn):
    return pl.pallas_call(
        _tc_a_body,
        grid=(NP // BLK,),
        in_specs=[
            pl.BlockSpec((BLK, F), lambda i: (i, 0)),
            pl.BlockSpec((F, F), lambda i: (0, 0)),
            pl.BlockSpec((8, F), lambda i: (0, 0)),
        ],
        out_specs=_PROJ_OUT_SPECS,
        out_shape=_PROJ_OUT_SHAPE,
    )(xp, wT, attn)


def _tc_mid(msg, esums, b2d, wT, attn):
    return pl.pallas_call(
        _tc_b_body,
        grid=(NP // BLK,),
        in_specs=[
            pl.BlockSpec((BLK, F), lambda i: (i, 0)),
            pl.BlockSpec((BLK, F), lambda i: (i, 0)),
            pl.BlockSpec((NWORK, BLK), lambda i: (0, i)),
            pl.BlockSpec((8, F), lambda i: (0, 0)),
            pl.BlockSpec((F, F), lambda i: (0, 0)),
            pl.BlockSpec((8, F), lambda i: (0, 0)),
        ],
        out_specs=_PROJ_OUT_SPECS,
        out_shape=_PROJ_OUT_SHAPE,
    )(msg[0], msg[1], esums, b2d, wT, attn)


def _tc_back(msg, esums, b2d):
    ngrid = (N + BLK - 1) // BLK
    return pl.pallas_call(
        _tc_c_body,
        grid=(ngrid,),
        in_specs=[
            pl.BlockSpec((BLK, F), lambda i: (i, 0)),
            pl.BlockSpec((BLK, F), lambda i: (i, 0)),
            pl.BlockSpec((NWORK, BLK), lambda i: (0, i)),
            pl.BlockSpec((8, F), lambda i: (0, 0)),
        ],
        out_specs=pl.BlockSpec((BLK, F), lambda i: (i, 0)),
        out_shape=jax.ShapeDtypeStruct((N, F), jnp.float32),
    )(msg[0], msg[1], esums, b2d)


def _sc_body(feat_hbm, elr_hbm, mstat_hbm, srcw_hbm, dstw_hbm,
             msg_hbm, esums_hbm,
             acc, el_v, er_v, esum_v, srcw_v, dstw_v, dstb_v, rows_v, m_v,
             gsem0, gsem1, gsem2, ssem0, ssem1, ssem2, wsem):
    gsems = (gsem0, gsem1, gsem2)
    ssems = (ssem0, ssem1, ssem2)
    c = lax.axis_index("c")
    s = lax.axis_index("s")
    wid = c * NS + s
    base_row = s * ROWS_PT
    n48 = jnp.where(s < NS - 1, ROWS_PT // BATCH, (NACC - 15 * ROWS_PT - 8)
                    // BATCH)

    def zrow(i, _):
        for cc in range(F // 16):
            rows_v[0, i, pl.ds(cc * 16, 16)] = jnp.zeros((16,), jnp.float32)
        return 0
    lax.fori_loop(0, BATCH, zrow, 0)

    def zacc(i, _):
        pltpu.sync_copy(rows_v.at[0],
                        acc.at[pl.ds(base_row + i * BATCH, BATCH)])
        return 0
    lax.fori_loop(0, n48, zacc, 0)
    pltpu.sync_copy(rows_v.at[0, pl.ds(0, 8)],
                    acc.at[pl.ds(base_row + n48 * BATCH, 8)])

    def zes(i, _):
        esum_v[pl.ds(i * 16, 16)] = jnp.zeros((16,), jnp.float32)
        return 0
    lax.fori_loop(0, NP // 16, zes, 0)

    pltpu.sync_copy(elr_hbm.at[0], el_v)
    pltpu.sync_copy(elr_hbm.at[1], er_v)
    pltpu.sync_copy(mstat_hbm.at[0], m_v.at[pl.ds(0, F)])
    pltpu.sync_copy(mstat_hbm.at[1], m_v.at[pl.ds(F, F)])
    mm = m_v[pl.ds(0, 16)] + m_v[pl.ds(F, 16)]
    big_m = jnp.where(mm >= 0, mm, NEG * mm)

    plsc.subcore_barrier()

    WW = KWIN * BATCH
    ebase = wid * PER_TILE

    NWIN = NBATCH // KWIN

    def fetch_window(wi):
        o = lax.rem(wi, 2) * WW
        g0 = ebase + wi * WW
        pltpu.async_copy(srcw_hbm.at[pl.ds(g0, WW)],
                         srcw_v.at[pl.ds(o, WW)], wsem)
        pltpu.async_copy(dstw_hbm.at[pl.ds(g0, WW)],
                         dstw_v.at[pl.ds(o, WW)], wsem)

    def wait_window():
        pltpu.make_async_copy(srcw_hbm.at[pl.ds(0, WW)],
                              srcw_v.at[pl.ds(0, WW)], wsem).wait()
        pltpu.make_async_copy(dstw_hbm.at[pl.ds(0, WW)],
                              dstw_v.at[pl.ds(0, WW)], wsem).wait()

    def woff(j):
        return (lax.rem(lax.div(j, KWIN), 2) * KWIN + lax.rem(j, KWIN)) * BATCH

    def start_gather(slot, j):
        pltpu.async_copy(feat_hbm.at[srcw_v.at[pl.ds(woff(j), BATCH)]],
                         rows_v.at[slot], gsems[slot])

    def wait_gather(slot):
        pltpu.make_async_copy(feat_hbm.at[srcw_v.at[pl.ds(0, BATCH)]],
                              rows_v.at[slot], gsems[slot]).wait()

    def wait_scatter(slot):
        pltpu.make_async_copy(rows_v.at[slot], acc.at[dstb_v.at[slot]],
                              ssems[slot]).wait()

    def compute(slot, j):
        o = woff(j)
        for g in range(BATCH // 16):
            sl = pl.ds(o + g * 16, 16)
            s16 = srcw_v[sl]
            d16 = dstw_v[sl]
            dstb_v[slot, pl.ds(g * 16, 16)] = d16
            e = plsc.load_gather(el_v, [s16]) + plsc.load_gather(er_v, [d16])
            e = jnp.where(e >= 0, e, NEG * e)
            ee = jnp.exp(e - big_m)
            plsc.addupdate_scatter(esum_v, [d16], ee)
            for k in range(16):
                eei = ee[k]
                edge = g * 16 + k
                for cc in range(F // 16):
                    csl = pl.ds(cc * 16, 16)
                    rows_v[slot, edge, csl] = rows_v[slot, edge, csl] * eei

    def scatter(slot, j):
        pltpu.async_copy(rows_v.at[slot], acc.at[dstb_v.at[slot]],
                         ssems[slot], add=True)

    def visit(v, b, first):
        p = v + (NBUF - 1)

        def prefetch():
            bp = (b + NBUF - 1) % NBUF
            rp = lax.rem(jnp.int32(p), KWIN)
            pl.when(rp == 0)(wait_window)

            def fetch_next():
                wi = lax.div(jnp.int32(p) - 2, KWIN) + 1
                pl.when(wi < NWIN)(lambda: fetch_window(wi))
            pl.when(rp == 2)(fetch_next)
            if not first:
                wait_scatter(bp)
            start_gather(bp, p)
        if isinstance(v, int):
            prefetch()
        else:
            pl.when(p < NBATCH)(prefetch)
        wait_gather(b)
        compute(b, v)
        scatter(b, v)

    fetch_window(0)
    wait_window()
    for j in range(NBUF - 1):
        start_gather(j, j)
    for j in range(NBUF):
        visit(j, j, j == 0)

    def round_body(r, _):
        v = r * NBUF
        for i in range(NBUF):
            visit(v + i, i, False)
        return 0
    lax.fori_loop(1, NBATCH // NBUF, round_body, 0)
    for j in range(NBUF):
        wait_scatter(j)
    plsc.subcore_barrier()

    def wb(i, _):
        r0 = base_row + i * BATCH
        pltpu.sync_copy(acc.at[pl.ds(r0, BATCH)], rows_v.at[0])
        pltpu.sync_copy(rows_v.at[0], msg_hbm.at[c, pl.ds(r0, BATCH)])
        return 0
    lax.fori_loop(0, n48, wb, 0)
    rt = base_row + n48 * BATCH
    pltpu.sync_copy(acc.at[pl.ds(rt, 8)], rows_v.at[0, pl.ds(0, 8)])
    pltpu.sync_copy(rows_v.at[0, pl.ds(0, 8)], msg_hbm.at[c, pl.ds(rt, 8)])
    pltpu.sync_copy(esum_v, esums_hbm.at[wid])


def _sc_agg(featE, elr, mstat, srcw, dstw):
    mesh = plsc.VectorSubcoreMesh(core_axis_name="c", subcore_axis_name="s",
                                  num_cores=NC, num_subcores=NS)
    run = pl.kernel(
        _sc_body,
        out_type=[
            jax.ShapeDtypeStruct((NC, NACC, F), jnp.float32),
            jax.ShapeDtypeStruct((NWORK, NP), jnp.float32),
        ],
        mesh=mesh,
        compiler_params=pltpu.CompilerParams(needs_layout_passes=False),
        scratch_types=[
            pltpu.VMEM_SHARED((NACC, F), jnp.float32),
            pltpu.VMEM((NP,), jnp.float32),
            pltpu.VMEM((NP,), jnp.float32),
            pltpu.VMEM((NP,), jnp.float32),
            pltpu.VMEM((2 * KWIN * BATCH,), jnp.int32),
            pltpu.VMEM((2 * KWIN * BATCH,), jnp.int32),
            pltpu.VMEM((NBUF, BATCH), jnp.int32),
            pltpu.VMEM((NBUF, BATCH, F), jnp.float32),
            pltpu.VMEM((2 * F,), jnp.float32),
            pltpu.SemaphoreType.DMA,
            pltpu.SemaphoreType.DMA,
            pltpu.SemaphoreType.DMA,
            pltpu.SemaphoreType.DMA,
            pltpu.SemaphoreType.DMA,
            pltpu.SemaphoreType.DMA,
            pltpu.SemaphoreType.DMA,
        ],
    )
    return run(featE, elr, mstat, srcw, dstw)


def kernel(x, edge_index, W1, attn_l1, attn_r1, b1, W2, attn_l2, attn_r2, b2):
    f32 = jnp.float32
    xp = jnp.pad(x.astype(f32), ((0, NP - N), (0, 0)))
    loop = jnp.arange(N, dtype=jnp.int32)
    padv = jnp.full((E_PAD - E_TOT,), N, jnp.int32)
    srcw = jnp.concatenate([edge_index[0], loop, padv])
    dstw = jnp.concatenate([edge_index[1], loop, padv])

    def attn_pack(al, ar):
        a = jnp.zeros((8, F), f32)
        return a.at[0].set(al[0]).at[1].set(ar[0])

    attn1 = attn_pack(attn_l1, attn_r1)
    attn2 = attn_pack(attn_l2, attn_r2)
    b1d = jnp.broadcast_to(b1.reshape(1, F), (8, F))
    b2d = jnp.broadcast_to(b2.reshape(1, F), (8, F))

    featE1, elr1, mstat1 = _tc_front(xp, W1.T, attn1)
    msg1, esum1 = _sc_agg(featE1, elr1, mstat1, srcw, dstw)
    featE2, elr2, mstat2 = _tc_mid(msg1, esum1, b1d, W2.T, attn2)
    msg2, esum2 = _sc_agg(featE2, elr2, mstat2, srcw, dstw)
    return _tc_back(msg2, esum2, b2d)
